# baseline (speedup 1.0000x reference)
import functools

import jax
import jax.numpy as jnp
from jax import lax
from jax.experimental import pallas as pl
from jax.experimental.pallas import tpu as pltpu
from jax.experimental.pallas import tpu_sc as plsc

B = 16384
F = 32
NC = 2
NS = 16
NW = NC * NS
BPW = B // NW
CHUNK = 128
NCHUNK = BPW // CHUNK
L = 16


def _mf_body(uid_hbm, iid_hbm, uf_hbm, if_hbm, ub_hbm, ib_hbm, out_hbm,
             uid_v, iid_v, urows_v, irows_v, ub_v, ib_v, out_v, sem):
    wid = lax.axis_index("s") * NC + lax.axis_index("c")
    base = wid * BPW

    pltpu.sync_copy(uid_hbm.at[pl.ds(base, BPW)], uid_v)
    pltpu.sync_copy(iid_hbm.at[pl.ds(base, BPW)], iid_v)

    copies = []
    for j in range(NCHUNK):
        idx_u = uid_v.at[pl.ds(j * CHUNK, CHUNK)]
        idx_i = iid_v.at[pl.ds(j * CHUNK, CHUNK)]
        dst = pl.ds(j * CHUNK, CHUNK)
        copies.append(pltpu.async_copy(uf_hbm.at[idx_u], urows_v.at[dst], sem))
        copies.append(pltpu.async_copy(if_hbm.at[idx_i], irows_v.at[dst], sem))
        copies.append(pltpu.async_copy(ub_hbm.at[idx_u], ub_v.at[dst], sem))
        copies.append(pltpu.async_copy(ib_hbm.at[idx_i], ib_v.at[dst], sem))
    for c in copies:
        c.wait()

    def block(blk, carry):
        rows = blk * L + lax.iota(jnp.int32, L)
        acc = plsc.load_gather(ub_v, [rows])
        acc = acc + plsc.load_gather(ib_v, [rows])
        for f in range(F):
            cf = jnp.full((L,), f, jnp.int32)
            u = plsc.load_gather(urows_v, [rows, cf])
            v = plsc.load_gather(irows_v, [rows, cf])
            acc = acc + u * v
        plsc.store_scatter(out_v, [rows], acc)
        return carry

    lax.fori_loop(0, BPW // L, block, 0)

    pltpu.sync_copy(out_v, out_hbm.at[pl.ds(base, BPW)])


@jax.jit
def _mf_call(user_ids, item_ids, user_factors, item_factors,
             user_biases, item_biases):
    mesh = plsc.VectorSubcoreMesh(core_axis_name="c", subcore_axis_name="s")
    f = functools.partial(
        pl.kernel,
        mesh=mesh,
        out_type=jax.ShapeDtypeStruct((B,), jnp.float32),
        compiler_params=pltpu.CompilerParams(
            needs_layout_passes=False, use_tc_tiling_on_sc=False),
        scratch_types=[
            pltpu.VMEM((BPW,), jnp.int32),
            pltpu.VMEM((BPW,), jnp.int32),
            pltpu.VMEM((BPW, F), jnp.float32),
            pltpu.VMEM((BPW, F), jnp.float32),
            pltpu.VMEM((BPW,), jnp.float32),
            pltpu.VMEM((BPW,), jnp.float32),
            pltpu.VMEM((BPW,), jnp.float32),
            pltpu.SemaphoreType.DMA,
        ],
    )(_mf_body)
    return f(user_ids, item_ids, user_factors, item_factors,
             user_biases.reshape(-1), item_biases.reshape(-1))


def kernel(user_ids, item_ids, user_factors, item_factors,
           user_biases, item_biases, global_bias):
    out = _mf_call(user_ids, item_ids, user_factors, item_factors,
                   user_biases, item_biases)
    return out + global_bias
